# baseline (speedup 1.0000x reference)
import jax
import jax.numpy as jnp
from jax.experimental import pallas as pl
from jax.experimental.pallas import tpu as pltpu

_H, _W, _KH, _KW = 768, 768, 3, 3
_P = _KH * _KW
_R = _P * _H
_CN = (192, 384, 768, 2304, 2304, 768, 192)
_CO = (0, 192, 576, 1344, 3648, 5952, 6720)
_NCH = len(_CN)
_MAXC = max(_CN)
_NBUF = 3


def _mul_body(m_hbm, s_hbm, o_hbm, m_buf, *scr):
    in_bufs = scr[0:_NBUF]
    out_bufs = scr[_NBUF:2 * _NBUF]
    msem = scr[2 * _NBUF]
    isems = scr[2 * _NBUF + 1:2 * _NBUF + 1 + _NBUF]
    osems = scr[2 * _NBUF + 1 + _NBUF:2 * _NBUF + 1 + 2 * _NBUF]

    def start_in(i):
        pltpu.make_async_copy(
            s_hbm.at[pl.ds(_CO[i], _CN[i])],
            in_bufs[i % _NBUF].at[pl.ds(0, _CN[i])], isems[i % _NBUF]
        ).start()

    def wait_in(i):
        pltpu.make_async_copy(
            s_hbm.at[pl.ds(_CO[i], _CN[i])],
            in_bufs[i % _NBUF].at[pl.ds(0, _CN[i])], isems[i % _NBUF]
        ).wait()

    def start_out(i):
        pltpu.make_async_copy(
            out_bufs[i % _NBUF].at[pl.ds(0, _CN[i])],
            o_hbm.at[pl.ds(_CO[i], _CN[i])], osems[i % _NBUF]
        ).start()

    def wait_out(i):
        pltpu.make_async_copy(
            out_bufs[i % _NBUF].at[pl.ds(0, _CN[i])],
            o_hbm.at[pl.ds(_CO[i], _CN[i])], osems[i % _NBUF]
        ).wait()

    pltpu.make_async_copy(m_hbm, m_buf, msem).start()
    for i in range(_NBUF):
        start_in(i)
    pltpu.make_async_copy(m_hbm, m_buf, msem).wait()
    for i in range(_NCH):
        if i >= _NBUF:
            wait_out(i - _NBUF)
        wait_in(i)
        pos = 0
        while pos < _CN[i]:
            a = (_CO[i] + pos) % _H
            m = min(_H - a, _CN[i] - pos)
            out_bufs[i % _NBUF][pl.ds(pos, m)] = (
                m_buf[pl.ds(a, m), :] * in_bufs[i % _NBUF][pl.ds(pos, m)]
            )
            pos += m
        start_out(i)
        if i + _NBUF < _NCH:
            start_in(i + _NBUF)
    for i in range(max(0, _NCH - _NBUF), _NCH):
        wait_out(i)


def kernel(mask, s_tensor):
    st = jnp.transpose(s_tensor, (2, 3, 0, 1)).reshape(_R, _W)
    out = pl.pallas_call(
        _mul_body,
        in_specs=[
            pl.BlockSpec(memory_space=pltpu.MemorySpace.HBM),
            pl.BlockSpec(memory_space=pltpu.MemorySpace.HBM),
        ],
        out_specs=pl.BlockSpec(memory_space=pltpu.MemorySpace.HBM),
        out_shape=jax.ShapeDtypeStruct((_R, _W), jnp.float32),
        scratch_shapes=(
            [pltpu.VMEM((_H, _W), jnp.float32)]
            + [pltpu.VMEM((_MAXC, _W), jnp.float32) for _ in range(2 * _NBUF)]
            + [pltpu.SemaphoreType.DMA for _ in range(2 * _NBUF + 1)]
        ),
    )(mask, st)
    return out.reshape(_KH, _KW, _H, _W).transpose(2, 3, 0, 1)


# speedup vs baseline: 1.0135x; 1.0135x over previous
import jax
import jax.numpy as jnp
from jax.experimental import pallas as pl
from jax.experimental.pallas import tpu as pltpu

_H, _W, _KH, _KW = 768, 768, 3, 3
_P = _KH * _KW
_R = _P * _H
_CN = (128, 256, 512, 1024, 1536, 1536, 1024, 512, 256, 128)
_CO = (0, 128, 384, 896, 1920, 3456, 4992, 6016, 6528, 6784)
_NCH = len(_CN)
_MAXC = max(_CN)
_NBUF = 4


def _mul_body(m_hbm, s_hbm, o_hbm, m_buf, *scr):
    in_bufs = scr[0:_NBUF]
    out_bufs = scr[_NBUF:2 * _NBUF]
    msem = scr[2 * _NBUF]
    isems = scr[2 * _NBUF + 1:2 * _NBUF + 1 + _NBUF]
    osems = scr[2 * _NBUF + 1 + _NBUF:2 * _NBUF + 1 + 2 * _NBUF]

    def start_in(i):
        pltpu.make_async_copy(
            s_hbm.at[pl.ds(_CO[i], _CN[i])],
            in_bufs[i % _NBUF].at[pl.ds(0, _CN[i])], isems[i % _NBUF]
        ).start()

    def wait_in(i):
        pltpu.make_async_copy(
            s_hbm.at[pl.ds(_CO[i], _CN[i])],
            in_bufs[i % _NBUF].at[pl.ds(0, _CN[i])], isems[i % _NBUF]
        ).wait()

    def start_out(i):
        pltpu.make_async_copy(
            out_bufs[i % _NBUF].at[pl.ds(0, _CN[i])],
            o_hbm.at[pl.ds(_CO[i], _CN[i])], osems[i % _NBUF]
        ).start()

    def wait_out(i):
        pltpu.make_async_copy(
            out_bufs[i % _NBUF].at[pl.ds(0, _CN[i])],
            o_hbm.at[pl.ds(_CO[i], _CN[i])], osems[i % _NBUF]
        ).wait()

    pltpu.make_async_copy(m_hbm, m_buf, msem).start()
    for i in range(_NBUF):
        start_in(i)
    pltpu.make_async_copy(m_hbm, m_buf, msem).wait()
    for i in range(_NCH):
        if i >= _NBUF:
            wait_out(i - _NBUF)
        wait_in(i)
        pos = 0
        while pos < _CN[i]:
            a = (_CO[i] + pos) % _H
            m = min(_H - a, _CN[i] - pos)
            out_bufs[i % _NBUF][pl.ds(pos, m)] = (
                m_buf[pl.ds(a, m), :] * in_bufs[i % _NBUF][pl.ds(pos, m)]
            )
            pos += m
        start_out(i)
        if i + _NBUF < _NCH:
            start_in(i + _NBUF)
    for i in range(max(0, _NCH - _NBUF), _NCH):
        wait_out(i)


def kernel(mask, s_tensor):
    st = jnp.transpose(s_tensor, (2, 3, 0, 1)).reshape(_R, _W)
    out = pl.pallas_call(
        _mul_body,
        in_specs=[
            pl.BlockSpec(memory_space=pltpu.MemorySpace.HBM),
            pl.BlockSpec(memory_space=pltpu.MemorySpace.HBM),
        ],
        out_specs=pl.BlockSpec(memory_space=pltpu.MemorySpace.HBM),
        out_shape=jax.ShapeDtypeStruct((_R, _W), jnp.float32),
        scratch_shapes=(
            [pltpu.VMEM((_H, _W), jnp.float32)]
            + [pltpu.VMEM((_MAXC, _W), jnp.float32) for _ in range(2 * _NBUF)]
            + [pltpu.SemaphoreType.DMA for _ in range(2 * _NBUF + 1)]
        ),
    )(mask, st)
    return out.reshape(_KH, _KW, _H, _W).transpose(2, 3, 0, 1)
